# Initial kernel scaffold; baseline (speedup 1.0000x reference)
import jax
import jax.numpy as jnp
from jax.experimental import pallas as pl


def kernel(xyz0, xyz1, xyz2, x0, x1, x2):
    raise NotImplementedError("write your pallas kernel here")


# speedup vs baseline: 32.3910x; 32.3910x over previous
import jax
import jax.numpy as jnp
from jax.experimental import pallas as pl


def _propagate_body(D1, S, q_ref, c_ref, p1_ref, p2_ref, o_ref):
    q = q_ref[0]
    c = c_ref[0]
    qq = jnp.sum(q * q, axis=1)
    cc = jnp.sum(c * c, axis=1)
    dot = jax.lax.dot_general(q, c, (((1,), (1,)), ((), ())),
                              preferred_element_type=jnp.float32)
    d = -2.0 * dot + qq[:, None] + cc[None, :]

    iota = jax.lax.broadcasted_iota(jnp.int32, d.shape, 1)
    big = jnp.float32(jnp.inf)
    d1 = jnp.min(d, axis=1)
    i1 = jnp.min(jnp.where(d == d1[:, None], iota, S), axis=1)
    dm = jnp.where(iota == i1[:, None], big, d)
    d2 = jnp.min(dm, axis=1)
    i2 = jnp.min(jnp.where(dm == d2[:, None], iota, S), axis=1)
    dm2 = jnp.where(iota == i2[:, None], big, dm)
    d3 = jnp.min(dm2, axis=1)
    i3 = jnp.min(jnp.where(dm2 == d3[:, None], iota, S), axis=1)

    r1 = 1.0 / (d1 + 1e-8)
    r2 = 1.0 / (d2 + 1e-8)
    r3 = 1.0 / (d3 + 1e-8)
    norm = r1 + r2 + r3
    w = (jnp.where(iota == i1[:, None], (r1 / norm)[:, None], 0.0)
         + jnp.where(iota == i2[:, None], (r2 / norm)[:, None], 0.0)
         + jnp.where(iota == i3[:, None], (r3 / norm)[:, None], 0.0))

    interp = jax.lax.dot_general(p2_ref[0], w, (((1,), (1,)), ((), ())),
                                 preferred_element_type=jnp.float32)
    o_ref[0, :D1, :] = p1_ref[0]
    o_ref[0, D1:, :] = interp


def _propagate(xyz1, xyz2, points1, points2, n_tile):
    B, N, _ = xyz1.shape
    S = xyz2.shape[1]
    D1 = points1.shape[1]
    D2 = points2.shape[1]
    import functools
    body = functools.partial(_propagate_body, D1, S)
    return pl.pallas_call(
        body,
        grid=(B, N // n_tile),
        in_specs=[
            pl.BlockSpec((1, n_tile, 3), lambda b, n: (b, n, 0)),
            pl.BlockSpec((1, S, 3), lambda b, n: (b, 0, 0)),
            pl.BlockSpec((1, D1, n_tile), lambda b, n: (b, 0, n)),
            pl.BlockSpec((1, D2, S), lambda b, n: (b, 0, 0)),
        ],
        out_specs=pl.BlockSpec((1, D1 + D2, n_tile), lambda b, n: (b, 0, n)),
        out_shape=jax.ShapeDtypeStruct((B, D1 + D2, N), jnp.float32),
    )(xyz1, xyz2, points1, points2)


def kernel(xyz0, xyz1, xyz2, x0, x1, x2):
    y1 = _propagate(xyz1, xyz2, x1, x2, 256)
    out = _propagate(xyz0, xyz1, x0, y1, 256)
    return out
